# speedup vs baseline: 1.1254x; 1.1254x over previous
import functools
import math

import jax
import jax.numpy as jnp
from jax import lax
from jax.experimental import pallas as pl
from jax.experimental.pallas import tpu as pltpu
from jax.experimental.pallas import tpu_sc as plsc

D_MODEL = 1024
SCALE = math.sqrt(D_MODEL)
LANES = 16

_info = plsc.get_sparse_core_info()
NUM_CORES = _info.num_cores
NUM_SUBCORES = _info.num_subcores
NUM_WORKERS = NUM_CORES * NUM_SUBCORES


def _make_kernel(B: int):
    assert B % NUM_WORKERS == 0
    b_per_w = B // NUM_WORKERS
    CHUNK = 32
    assert b_per_w % CHUNK == 0
    n_chunks = b_per_w // CHUNK

    mesh = plsc.VectorSubcoreMesh(core_axis_name="c", subcore_axis_name="s")

    @functools.partial(
        pl.kernel,
        mesh=mesh,
        out_type=jax.ShapeDtypeStruct((B, D_MODEL), jnp.float32),
        scratch_types=[
            pltpu.VMEM((b_per_w,), jnp.int32),
            pltpu.VMEM((CHUNK, D_MODEL), jnp.float32),
            pltpu.SemaphoreType.DMA,
        ],
    )
    def emb_kernel(x_hbm, table_hbm, out_hbm, idx_v, buf, sem):
        wid = lax.axis_index("s") * NUM_CORES + lax.axis_index("c")
        base = wid * b_per_w
        pltpu.sync_copy(x_hbm.at[pl.ds(base, b_per_w)], idx_v)

        @pl.loop(0, n_chunks)
        def _chunk_loop(j):
            pltpu.async_copy(
                table_hbm.at[idx_v.at[pl.ds(j * CHUNK, CHUNK)]], buf, sem
            ).wait()

            @pl.loop(0, CHUNK)
            def _row_loop(r):
                @pl.loop(0, D_MODEL // LANES, unroll=8)
                def _vec_loop(k):
                    sl = pl.ds(k * LANES, LANES)
                    buf[r, sl] = buf[r, sl] * SCALE

            pltpu.sync_copy(buf, out_hbm.at[pl.ds(base + j * CHUNK, CHUNK)])

    return emb_kernel


@jax.jit
def kernel(x, table):
    B = x.shape[0] * x.shape[1]
    flat_idx = x.reshape(B).astype(jnp.int32)
    out = _make_kernel(B)(flat_idx, table)
    return out.reshape(x.shape[0], x.shape[1], D_MODEL)
